# baseline (speedup 1.0000x reference)
import functools

import jax
import jax.numpy as jnp
from jax import lax
from jax.experimental import pallas as pl
from jax.experimental.pallas import tpu as pltpu
from jax.experimental.pallas import tpu_sc as plsc

import kernel_r3_impl


def kernel(viewer_ids, movie_ids, viewer_table, movie_table):
    B = viewer_ids.shape[0]
    iota = lax.iota(jnp.int32, B)
    vs, vp = lax.sort_key_val(viewer_ids, iota)
    ms, mp = lax.sort_key_val(movie_ids, iota)
    out = kernel_r3_impl.kernel(viewer_ids, movie_ids, viewer_table, movie_table)
    return out + 0.0 * (vs[0] + vp[0] + ms[0] + mp[0]).astype(jnp.float32)


# speedup vs baseline: 2.3592x; 2.3592x over previous
import functools

import jax
import jax.numpy as jnp
from jax import lax
from jax.experimental import pallas as pl
from jax.experimental.pallas import tpu as pltpu
from jax.experimental.pallas import tpu_sc as plsc

_NC = 2
_NS = 16
_NW = _NC * _NS
_L = 16
_W = 128
_NB = 6


def _prep(ids, n_rows, b_per_w):
    B = ids.shape[0]
    iota = lax.iota(jnp.int32, B)
    sid, pos = lax.sort_key_val(ids, iota)
    ncut = (n_rows // _W) * _W
    tail = sid >= ncut
    blk1 = jnp.where(tail, 0, (sid >> 7) + 1)
    col = jnp.where(tail, sid - ncut, sid & (_W - 1))
    jloc = iota % b_per_w
    first = jnp.concatenate(
        [jnp.ones((1,), jnp.bool_), blk1[1:] != blk1[:-1]]
    ) | (jloc == 0)
    ordi = (
        jnp.cumsum(first.astype(jnp.int32).reshape(_NW, b_per_w), axis=1)
        .reshape(B) - 1
    )
    n_t = ordi.reshape(_NW, b_per_w)[:, -1] + 1
    word = col | (pos << 16)
    enc = blk1 | (jloc << 13)
    s1p = b_per_w + 16
    runs = jnp.full((_NW, s1p), b_per_w << 13, jnp.int32)
    runs = runs.at[iota // b_per_w, ordi].min(enc)
    runs = runs.at[:, s1p - 1].set(n_t)
    return word, runs.reshape(-1)


def _make_gather_kernel(B, K, b_per_w, NV, NM):
    mesh = plsc.VectorSubcoreMesh(core_axis_name="c", subcore_axis_name="s")
    S1 = b_per_w + 16

    @functools.partial(
        pl.kernel,
        mesh=mesh,
        compiler_params=pltpu.CompilerParams(needs_layout_passes=False),
        out_type=(
            jax.ShapeDtypeStruct((B * K,), jnp.float32),
            jax.ShapeDtypeStruct((B * K,), jnp.float32),
        ),
        scratch_types=[
            pltpu.VMEM((b_per_w,), jnp.int32),
            pltpu.VMEM((S1,), jnp.int32),
            pltpu.SMEM((b_per_w,), jnp.int32),
            pltpu.SMEM((S1,), jnp.int32),
            pltpu.VMEM((b_per_w * K,), jnp.float32),
        ]
        + [pltpu.VMEM((K, _W), jnp.float32) for _ in range(_NB)]
        + [pltpu.SemaphoreType.DMA for _ in range(_NB)]
        + [pltpu.SemaphoreType.DMA],
    )
    def gk(vtab_t, vtail, mtab_t, mtail, vword, vruns, mword, mruns,
           ustage, vstage,
           word_v, runs_v, word_s, runs_s, rowbuf,
           *rest):
        bufs = rest[:_NB]
        sems = rest[_NB:2 * _NB]
        sem_o = rest[2 * _NB]
        wid = lax.axis_index("s") * _NC + lax.axis_index("c")
        lanes = lax.iota(jnp.int32, _L)

        def run_table(tab, tail, word_hbm, runs_hbm, stage):
            pltpu.sync_copy(word_hbm.at[pl.ds(wid * b_per_w, b_per_w)], word_v)
            pltpu.sync_copy(runs_hbm.at[pl.ds(wid * S1, S1)], runs_v)

            def fill_w(g, _):
                v = word_v[pl.ds(g * _L, _L)]
                for l in range(_L):
                    word_s[g * _L + l] = v[l]
                return 0

            def fill_r(g, _):
                v = runs_v[pl.ds(g * _L, _L)]
                for l in range(_L):
                    runs_s[g * _L + l] = v[l]
                return 0

            lax.fori_loop(0, b_per_w // _L, fill_w, 0)
            lax.fori_loop(0, S1 // _L, fill_r, 0)
            n = runs_s[S1 - 1]

            def issue(f, s):
                e = runs_s[f]
                blk = e & 0x1FFF

                @pl.when(blk > 0)
                def _():
                    off = pl.multiple_of((blk - 1) * _W, _W)
                    pltpu.async_copy(
                        tab.at[:, pl.ds(off, _W)], bufs[s], sems[s]
                    )

                @pl.when(blk == 0)
                def _():
                    pltpu.async_copy(tail.at[:, pl.ds(0, _W)], bufs[s], sems[s])

            for s in range(_NB):
                @pl.when(s < n)
                def _(s=s):
                    issue(s, s)

            def gbody(g, _):
                for s in range(_NB):
                    f = g * _NB + s

                    @pl.when(f < n)
                    def _(f=f, s=s):
                        pltpu.make_async_copy(
                            tab.at[:, pl.ds(0, _W)], bufs[s], sems[s]
                        ).wait()
                        j0 = runs_s[f] >> 13
                        j1 = runs_s[f + 1] >> 13

                        def ebody(j, _):
                            w = word_s[j]
                            cv = lanes * 0 + (w & 0x7F)
                            g0 = plsc.load_gather(bufs[s], [lanes, cv])
                            g1 = plsc.load_gather(bufs[s], [lanes + _L, cv])
                            rowbuf[pl.ds(j * K, _L)] = g0
                            rowbuf[pl.ds(j * K + _L, _L)] = g1
                            return 0

                        lax.fori_loop(j0, j1, ebody, 0)

                        @pl.when(f + _NB < n)
                        def _():
                            issue(f + _NB, s)
                return 0

            lax.fori_loop(0, (n + _NB - 1) // _NB, gbody, 0)

            def obody(j, _):
                p = word_s[j] >> 16
                pltpu.async_copy(
                    rowbuf.at[pl.ds(j * K, K)],
                    stage.at[pl.ds(p * K, K)],
                    sem_o,
                )
                return 0

            lax.fori_loop(0, b_per_w, obody, 0)
            pltpu.make_async_copy(
                rowbuf, stage.at[pl.ds(0, b_per_w * K)], sem_o
            ).wait()

        run_table(vtab_t, vtail, vword, vruns, ustage)
        run_table(mtab_t, mtail, mword, mruns, vstage)

    return gk


def _make_dot_kernel(B, K, b_per_w):
    mesh = plsc.VectorSubcoreMesh(core_axis_name="c", subcore_axis_name="s")

    @functools.partial(
        pl.kernel,
        mesh=mesh,
        compiler_params=pltpu.CompilerParams(needs_layout_passes=False),
        out_type=jax.ShapeDtypeStruct((B,), jnp.float32),
        scratch_types=[
            pltpu.VMEM((b_per_w * K,), jnp.float32),
            pltpu.VMEM((b_per_w * K,), jnp.float32),
            pltpu.VMEM((b_per_w * _L,), jnp.float32),
            pltpu.VMEM((b_per_w,), jnp.float32),
        ],
    )
    def dk(ustage, vstage, out_hbm, ubuf, vbuf, partials, outv):
        wid = lax.axis_index("s") * _NC + lax.axis_index("c")
        base = wid * b_per_w
        pltpu.sync_copy(ustage.at[pl.ds(base * K, b_per_w * K)], ubuf)
        pltpu.sync_copy(vstage.at[pl.ds(base * K, b_per_w * K)], vbuf)
        lanes = lax.iota(jnp.int32, _L)

        def row_body(j, _):
            r0 = j * K
            p = ubuf[pl.ds(r0, _L)] * vbuf[pl.ds(r0, _L)]
            p = p + ubuf[pl.ds(r0 + _L, _L)] * vbuf[pl.ds(r0 + _L, _L)]
            partials[pl.ds(j * _L, _L)] = p
            return 0

        lax.fori_loop(0, b_per_w, row_body, 0)

        def red_body(g, _):
            bidx = g * (_L * _L) + lanes * _L
            acc = plsc.load_gather(partials, [bidx])
            for l in range(1, _L):
                acc = acc + plsc.load_gather(partials, [bidx + l])
            outv[pl.ds(g * _L, _L)] = acc
            return 0

        lax.fori_loop(0, b_per_w // _L, red_body, 0)
        pltpu.sync_copy(outv, out_hbm.at[pl.ds(base, b_per_w)])

    return dk


def kernel(viewer_ids, movie_ids, viewer_table, movie_table):
    B = viewer_ids.shape[0]
    K = viewer_table.shape[1]
    NV = viewer_table.shape[0]
    NM = movie_table.shape[0]
    b_per_w = B // _NW

    vword, vruns = _prep(viewer_ids, NV, b_per_w)
    mword, mruns = _prep(movie_ids, NM, b_per_w)

    vt = jnp.swapaxes(viewer_table, 0, 1)
    mt = jnp.swapaxes(movie_table, 0, 1)
    vcut = (NV // _W) * _W
    mcut = (NM // _W) * _W
    vtail = jnp.pad(
        jnp.swapaxes(viewer_table[vcut:], 0, 1), ((0, 0), (0, _W - (NV - vcut)))
    )
    mtail = jnp.pad(
        jnp.swapaxes(movie_table[mcut:], 0, 1), ((0, 0), (0, _W - (NM - mcut)))
    )

    gk = _make_gather_kernel(B, K, b_per_w, NV, NM)
    ustage, vstage = gk(vt, vtail, mt, mtail, vword, vruns, mword, mruns)
    dk = _make_dot_kernel(B, K, b_per_w)
    return dk(ustage, vstage)


# speedup vs baseline: 4.1889x; 1.7755x over previous
import functools

import jax
import jax.numpy as jnp
from jax import lax
from jax.experimental import pallas as pl
from jax.experimental.pallas import tpu as pltpu
from jax.experimental.pallas import tpu_sc as plsc

_NC = 2
_NS = 16
_NW = _NC * _NS
_L = 16
_W = 128
_NB = 6


def _prep(ids, n_rows, b_per_w):
    B = ids.shape[0]
    iota = lax.iota(jnp.int32, B)
    sid, pos = lax.sort_key_val(ids, iota)
    ncut = (n_rows // _W) * _W
    tail = sid >= ncut
    blk1 = jnp.where(tail, 0, (sid >> 7) + 1)
    col = jnp.where(tail, sid - ncut, sid & (_W - 1))
    jloc = iota % b_per_w
    first = jnp.concatenate(
        [jnp.ones((1,), jnp.bool_), blk1[1:] != blk1[:-1]]
    ) | (jloc == 0)
    ordi = (
        jnp.cumsum(first.astype(jnp.int32).reshape(_NW, b_per_w), axis=1)
        .reshape(B) - 1
    )
    n_t = ordi.reshape(_NW, b_per_w)[:, -1] + 1
    word = col | (pos << 16)
    pad_enc = b_per_w << 13
    enc = jnp.where(first, blk1 | (jloc << 13), pad_enc)
    key = jnp.where(first, ordi, jnp.int32(1 << 20)).reshape(_NW, b_per_w)
    _, senc = lax.sort_key_val(key, enc.reshape(_NW, b_per_w))
    runs = jnp.concatenate(
        [
            senc,
            jnp.full((_NW, 15), pad_enc, jnp.int32),
            n_t[:, None],
        ],
        axis=1,
    )
    return word, runs.reshape(-1)


def _make_gather_kernel(B, K, b_per_w, NV, NM):
    mesh = plsc.VectorSubcoreMesh(core_axis_name="c", subcore_axis_name="s")
    S1 = b_per_w + 16

    @functools.partial(
        pl.kernel,
        mesh=mesh,
        compiler_params=pltpu.CompilerParams(needs_layout_passes=False),
        out_type=(
            jax.ShapeDtypeStruct((B * K,), jnp.float32),
            jax.ShapeDtypeStruct((B * K,), jnp.float32),
        ),
        scratch_types=[
            pltpu.VMEM((b_per_w,), jnp.int32),
            pltpu.VMEM((S1,), jnp.int32),
            pltpu.SMEM((b_per_w,), jnp.int32),
            pltpu.SMEM((S1,), jnp.int32),
            pltpu.VMEM((b_per_w * K,), jnp.float32),
        ]
        + [pltpu.VMEM((K, _W), jnp.float32) for _ in range(_NB)]
        + [pltpu.SemaphoreType.DMA for _ in range(_NB)]
        + [pltpu.SemaphoreType.DMA],
    )
    def gk(vtab_t, vtail, mtab_t, mtail, vword, vruns, mword, mruns,
           ustage, vstage,
           word_v, runs_v, word_s, runs_s, rowbuf,
           *rest):
        bufs = rest[:_NB]
        sems = rest[_NB:2 * _NB]
        sem_o = rest[2 * _NB]
        wid = lax.axis_index("s") * _NC + lax.axis_index("c")
        lanes = lax.iota(jnp.int32, _L)

        def run_table(tab, tail, word_hbm, runs_hbm, stage):
            pltpu.sync_copy(word_hbm.at[pl.ds(wid * b_per_w, b_per_w)], word_v)
            pltpu.sync_copy(runs_hbm.at[pl.ds(wid * S1, S1)], runs_v)

            def fill_w(g, _):
                v = word_v[pl.ds(g * _L, _L)]
                for l in range(_L):
                    word_s[g * _L + l] = v[l]
                return 0

            def fill_r(g, _):
                v = runs_v[pl.ds(g * _L, _L)]
                for l in range(_L):
                    runs_s[g * _L + l] = v[l]
                return 0

            lax.fori_loop(0, b_per_w // _L, fill_w, 0)
            lax.fori_loop(0, S1 // _L, fill_r, 0)
            n = runs_s[S1 - 1]

            def issue(f, s):
                e = runs_s[f]
                blk = e & 0x1FFF

                @pl.when(blk > 0)
                def _():
                    off = pl.multiple_of((blk - 1) * _W, _W)
                    pltpu.async_copy(
                        tab.at[:, pl.ds(off, _W)], bufs[s], sems[s]
                    )

                @pl.when(blk == 0)
                def _():
                    pltpu.async_copy(tail.at[:, pl.ds(0, _W)], bufs[s], sems[s])

            for s in range(_NB):
                @pl.when(s < n)
                def _(s=s):
                    issue(s, s)

            def gbody(g, _):
                for s in range(_NB):
                    f = g * _NB + s

                    @pl.when(f < n)
                    def _(f=f, s=s):
                        pltpu.make_async_copy(
                            tab.at[:, pl.ds(0, _W)], bufs[s], sems[s]
                        ).wait()
                        j0 = runs_s[f] >> 13
                        j1 = runs_s[f + 1] >> 13

                        def ebody(j, _):
                            w = word_s[j]
                            cv = lanes * 0 + (w & 0x7F)
                            g0 = plsc.load_gather(bufs[s], [lanes, cv])
                            g1 = plsc.load_gather(bufs[s], [lanes + _L, cv])
                            rowbuf[pl.ds(j * K, _L)] = g0
                            rowbuf[pl.ds(j * K + _L, _L)] = g1
                            return 0

                        lax.fori_loop(j0, j1, ebody, 0)

                        @pl.when(f + _NB < n)
                        def _():
                            issue(f + _NB, s)
                return 0

            lax.fori_loop(0, (n + _NB - 1) // _NB, gbody, 0)

            def obody(j, _):
                p = word_s[j] >> 16
                pltpu.async_copy(
                    rowbuf.at[pl.ds(j * K, K)],
                    stage.at[pl.ds(p * K, K)],
                    sem_o,
                )
                return 0

            lax.fori_loop(0, b_per_w, obody, 0)
            pltpu.make_async_copy(
                rowbuf, stage.at[pl.ds(0, b_per_w * K)], sem_o
            ).wait()

        run_table(vtab_t, vtail, vword, vruns, ustage)
        run_table(mtab_t, mtail, mword, mruns, vstage)

    return gk


def _make_dot_kernel(B, K, b_per_w):
    mesh = plsc.VectorSubcoreMesh(core_axis_name="c", subcore_axis_name="s")

    @functools.partial(
        pl.kernel,
        mesh=mesh,
        compiler_params=pltpu.CompilerParams(needs_layout_passes=False),
        out_type=jax.ShapeDtypeStruct((B,), jnp.float32),
        scratch_types=[
            pltpu.VMEM((b_per_w * K,), jnp.float32),
            pltpu.VMEM((b_per_w * K,), jnp.float32),
            pltpu.VMEM((b_per_w * _L,), jnp.float32),
            pltpu.VMEM((b_per_w,), jnp.float32),
        ],
    )
    def dk(ustage, vstage, out_hbm, ubuf, vbuf, partials, outv):
        wid = lax.axis_index("s") * _NC + lax.axis_index("c")
        base = wid * b_per_w
        pltpu.sync_copy(ustage.at[pl.ds(base * K, b_per_w * K)], ubuf)
        pltpu.sync_copy(vstage.at[pl.ds(base * K, b_per_w * K)], vbuf)
        lanes = lax.iota(jnp.int32, _L)

        def row_body(j, _):
            r0 = j * K
            p = ubuf[pl.ds(r0, _L)] * vbuf[pl.ds(r0, _L)]
            p = p + ubuf[pl.ds(r0 + _L, _L)] * vbuf[pl.ds(r0 + _L, _L)]
            partials[pl.ds(j * _L, _L)] = p
            return 0

        lax.fori_loop(0, b_per_w, row_body, 0)

        def red_body(g, _):
            bidx = g * (_L * _L) + lanes * _L
            acc = plsc.load_gather(partials, [bidx])
            for l in range(1, _L):
                acc = acc + plsc.load_gather(partials, [bidx + l])
            outv[pl.ds(g * _L, _L)] = acc
            return 0

        lax.fori_loop(0, b_per_w // _L, red_body, 0)
        pltpu.sync_copy(outv, out_hbm.at[pl.ds(base, b_per_w)])

    return dk


def kernel(viewer_ids, movie_ids, viewer_table, movie_table):
    B = viewer_ids.shape[0]
    K = viewer_table.shape[1]
    NV = viewer_table.shape[0]
    NM = movie_table.shape[0]
    b_per_w = B // _NW

    vword, vruns = _prep(viewer_ids, NV, b_per_w)
    mword, mruns = _prep(movie_ids, NM, b_per_w)

    vt = jnp.swapaxes(viewer_table, 0, 1)
    mt = jnp.swapaxes(movie_table, 0, 1)
    vcut = (NV // _W) * _W
    mcut = (NM // _W) * _W
    vtail = jnp.pad(
        jnp.swapaxes(viewer_table[vcut:], 0, 1), ((0, 0), (0, _W - (NV - vcut)))
    )
    mtail = jnp.pad(
        jnp.swapaxes(movie_table[mcut:], 0, 1), ((0, 0), (0, _W - (NM - mcut)))
    )

    gk = _make_gather_kernel(B, K, b_per_w, NV, NM)
    ustage, vstage = gk(vt, vtail, mt, mtail, vword, vruns, mword, mruns)
    dk = _make_dot_kernel(B, K, b_per_w)
    return dk(ustage, vstage)


# speedup vs baseline: 4.4557x; 1.0637x over previous
import functools

import jax
import jax.numpy as jnp
from jax import lax
from jax.experimental import pallas as pl
from jax.experimental.pallas import tpu as pltpu
from jax.experimental.pallas import tpu_sc as plsc

_NC = 2
_NS = 16
_NW = _NC * _NS
_L = 16
_W = 128
_NB = 8


def _prep(ids, n_rows, b_per_w):
    B = ids.shape[0]
    iota = lax.iota(jnp.int32, B)
    sid, pos = lax.sort_key_val(ids, iota)
    ncut = (n_rows // _W) * _W
    tail = sid >= ncut
    blk1 = jnp.where(tail, 0, (sid >> 7) + 1)
    col = jnp.where(tail, sid - ncut, sid & (_W - 1))
    jloc = iota % b_per_w
    first = jnp.concatenate(
        [jnp.ones((1,), jnp.bool_), blk1[1:] != blk1[:-1]]
    ) | (jloc == 0)
    n_t = jnp.sum(first.reshape(_NW, b_per_w), axis=1, dtype=jnp.int32)
    word = col | (pos << 16)
    pad_enc = b_per_w << 13
    enc = jnp.where(first, blk1 | (jloc << 13), pad_enc)
    key = jnp.where(first, jloc, jnp.int32(1 << 20)).reshape(_NW, b_per_w)
    _, senc = lax.sort_key_val(key, enc.reshape(_NW, b_per_w))
    runs = jnp.concatenate(
        [
            senc,
            jnp.full((_NW, 15), pad_enc, jnp.int32),
            n_t[:, None],
        ],
        axis=1,
    )
    return word, runs.reshape(-1)


def _make_gather_kernel(B, K, b_per_w, NV, NM):
    mesh = plsc.VectorSubcoreMesh(core_axis_name="c", subcore_axis_name="s")
    S1 = b_per_w + 16

    @functools.partial(
        pl.kernel,
        mesh=mesh,
        compiler_params=pltpu.CompilerParams(needs_layout_passes=False),
        out_type=(
            jax.ShapeDtypeStruct((B * K,), jnp.float32),
            jax.ShapeDtypeStruct((B * K,), jnp.float32),
        ),
        scratch_types=[
            pltpu.VMEM((b_per_w,), jnp.int32),
            pltpu.VMEM((S1,), jnp.int32),
            pltpu.SMEM((b_per_w,), jnp.int32),
            pltpu.SMEM((S1,), jnp.int32),
            pltpu.VMEM((b_per_w * K,), jnp.float32),
        ]
        + [pltpu.VMEM((K, _W), jnp.float32) for _ in range(_NB)]
        + [pltpu.SemaphoreType.DMA for _ in range(_NB)]
        + [pltpu.SemaphoreType.DMA],
    )
    def gk(vtab_t, vtail, mtab_t, mtail, vword, vruns, mword, mruns,
           ustage, vstage,
           word_v, runs_v, word_s, runs_s, rowbuf,
           *rest):
        bufs = rest[:_NB]
        sems = rest[_NB:2 * _NB]
        sem_o = rest[2 * _NB]
        wid = lax.axis_index("s") * _NC + lax.axis_index("c")
        lanes = lax.iota(jnp.int32, _L)

        def run_table(tab, tail, word_hbm, runs_hbm, stage):
            pltpu.sync_copy(word_hbm.at[pl.ds(wid * b_per_w, b_per_w)], word_v)
            pltpu.sync_copy(runs_hbm.at[pl.ds(wid * S1, S1)], runs_v)

            def fill_w(g, _):
                v = word_v[pl.ds(g * _L, _L)]
                for l in range(_L):
                    word_s[g * _L + l] = v[l]
                return 0

            def fill_r(g, _):
                v = runs_v[pl.ds(g * _L, _L)]
                for l in range(_L):
                    runs_s[g * _L + l] = v[l]
                return 0

            lax.fori_loop(0, b_per_w // _L, fill_w, 0)
            lax.fori_loop(0, S1 // _L, fill_r, 0)
            n = runs_s[S1 - 1]

            def issue(f, s):
                e = runs_s[f]
                blk = e & 0x1FFF

                @pl.when(blk > 0)
                def _():
                    off = pl.multiple_of((blk - 1) * _W, _W)
                    pltpu.async_copy(
                        tab.at[:, pl.ds(off, _W)], bufs[s], sems[s]
                    )

                @pl.when(blk == 0)
                def _():
                    pltpu.async_copy(tail.at[:, pl.ds(0, _W)], bufs[s], sems[s])

            for s in range(_NB):
                @pl.when(s < n)
                def _(s=s):
                    issue(s, s)

            def gbody(g, _):
                for s in range(_NB):
                    f = g * _NB + s

                    @pl.when(f < n)
                    def _(f=f, s=s):
                        pltpu.make_async_copy(
                            tab.at[:, pl.ds(0, _W)], bufs[s], sems[s]
                        ).wait()
                        j0 = runs_s[f] >> 13
                        j1 = runs_s[f + 1] >> 13

                        def ebody(j, _):
                            w = word_s[j]
                            cv = lanes * 0 + (w & 0x7F)
                            g0 = plsc.load_gather(bufs[s], [lanes, cv])
                            g1 = plsc.load_gather(bufs[s], [lanes + _L, cv])
                            rowbuf[pl.ds(j * K, _L)] = g0
                            rowbuf[pl.ds(j * K + _L, _L)] = g1
                            return 0

                        lax.fori_loop(j0, j1, ebody, 0)

                        @pl.when(f + _NB < n)
                        def _():
                            issue(f + _NB, s)
                return 0

            lax.fori_loop(0, (n + _NB - 1) // _NB, gbody, 0)

            def obody(j, _):
                p = word_s[j] >> 16
                pltpu.async_copy(
                    rowbuf.at[pl.ds(j * K, K)],
                    stage.at[pl.ds(p * K, K)],
                    sem_o,
                )
                return 0

            lax.fori_loop(0, b_per_w, obody, 0)
            pltpu.make_async_copy(
                rowbuf, stage.at[pl.ds(0, b_per_w * K)], sem_o
            ).wait()

        run_table(vtab_t, vtail, vword, vruns, ustage)
        run_table(mtab_t, mtail, mword, mruns, vstage)

    return gk


def _make_dot_kernel(B, K, b_per_w):
    mesh = plsc.VectorSubcoreMesh(core_axis_name="c", subcore_axis_name="s")

    @functools.partial(
        pl.kernel,
        mesh=mesh,
        compiler_params=pltpu.CompilerParams(needs_layout_passes=False),
        out_type=jax.ShapeDtypeStruct((B,), jnp.float32),
        scratch_types=[
            pltpu.VMEM((b_per_w * K,), jnp.float32),
            pltpu.VMEM((b_per_w * K,), jnp.float32),
            pltpu.VMEM((b_per_w * _L,), jnp.float32),
            pltpu.VMEM((b_per_w,), jnp.float32),
        ],
    )
    def dk(ustage, vstage, out_hbm, ubuf, vbuf, partials, outv):
        wid = lax.axis_index("s") * _NC + lax.axis_index("c")
        base = wid * b_per_w
        pltpu.sync_copy(ustage.at[pl.ds(base * K, b_per_w * K)], ubuf)
        pltpu.sync_copy(vstage.at[pl.ds(base * K, b_per_w * K)], vbuf)
        lanes = lax.iota(jnp.int32, _L)

        def row_body(j, _):
            r0 = j * K
            p = ubuf[pl.ds(r0, _L)] * vbuf[pl.ds(r0, _L)]
            p = p + ubuf[pl.ds(r0 + _L, _L)] * vbuf[pl.ds(r0 + _L, _L)]
            partials[pl.ds(j * _L, _L)] = p
            return 0

        lax.fori_loop(0, b_per_w, row_body, 0)

        def red_body(g, _):
            bidx = g * (_L * _L) + lanes * _L
            acc = plsc.load_gather(partials, [bidx])
            for l in range(1, _L):
                acc = acc + plsc.load_gather(partials, [bidx + l])
            outv[pl.ds(g * _L, _L)] = acc
            return 0

        lax.fori_loop(0, b_per_w // _L, red_body, 0)
        pltpu.sync_copy(outv, out_hbm.at[pl.ds(base, b_per_w)])

    return dk


def kernel(viewer_ids, movie_ids, viewer_table, movie_table):
    B = viewer_ids.shape[0]
    K = viewer_table.shape[1]
    NV = viewer_table.shape[0]
    NM = movie_table.shape[0]
    b_per_w = B // _NW

    vword, vruns = _prep(viewer_ids, NV, b_per_w)
    mword, mruns = _prep(movie_ids, NM, b_per_w)

    vt = jnp.swapaxes(viewer_table, 0, 1)
    mt = jnp.swapaxes(movie_table, 0, 1)
    vcut = (NV // _W) * _W
    mcut = (NM // _W) * _W
    vtail = jnp.pad(
        jnp.swapaxes(viewer_table[vcut:], 0, 1), ((0, 0), (0, _W - (NV - vcut)))
    )
    mtail = jnp.pad(
        jnp.swapaxes(movie_table[mcut:], 0, 1), ((0, 0), (0, _W - (NM - mcut)))
    )

    gk = _make_gather_kernel(B, K, b_per_w, NV, NM)
    ustage, vstage = gk(vt, vtail, mt, mtail, vword, vruns, mword, mruns)
    dk = _make_dot_kernel(B, K, b_per_w)
    return dk(ustage, vstage)


# speedup vs baseline: 4.6650x; 1.0470x over previous
import functools

import jax
import jax.numpy as jnp
from jax import lax
from jax.experimental import pallas as pl
from jax.experimental.pallas import tpu as pltpu
from jax.experimental.pallas import tpu_sc as plsc

_NC = 2
_NS = 16
_NW = _NC * _NS
_L = 16
_W = 128
_NB = 8


def _prep(viewer_ids, movie_ids, nv, nm, b_per_w):
    B = viewer_ids.shape[0]
    iota2 = lax.iota(jnp.int32, 2 * B)
    both = jnp.concatenate([viewer_ids, movie_ids + (1 << 20)])
    skey, sval = lax.sort_key_val(both, iota2)
    sid = skey.reshape(2, B) - jnp.array([[0], [1 << 20]], jnp.int32)
    pos = sval.reshape(2, B) & (B - 1)
    ncut = jnp.array([[(nv // _W) * _W], [(nm // _W) * _W]], jnp.int32)
    tail = sid >= ncut
    blk1 = jnp.where(tail, 0, (sid >> 7) + 1)
    col = jnp.where(tail, sid - ncut, sid & (_W - 1))
    jloc = (iota2 % b_per_w).reshape(2, B)
    first = jnp.concatenate(
        [jnp.ones((2, 1), jnp.bool_), blk1[:, 1:] != blk1[:, :-1]], axis=1
    ) | (jloc == 0)
    n_t = jnp.sum(
        first.reshape(2 * _NW, b_per_w), axis=1, dtype=jnp.int32
    )
    word = (col | (pos << 16)).reshape(-1)
    pad_enc = b_per_w << 13
    enc = jnp.where(first, blk1 | (jloc << 13), pad_enc)
    key = jnp.where(first, jloc, jnp.int32(1 << 20)).reshape(
        2 * _NW, b_per_w
    )
    _, senc = lax.sort_key_val(key, enc.reshape(2 * _NW, b_per_w))
    runs = jnp.concatenate(
        [
            senc,
            jnp.full((2 * _NW, 15), pad_enc, jnp.int32),
            n_t[:, None],
        ],
        axis=1,
    )
    return word, runs.reshape(-1)


def _make_gather_kernel(B, K, b_per_w, NV, NM):
    mesh = plsc.VectorSubcoreMesh(core_axis_name="c", subcore_axis_name="s")
    S1 = b_per_w + 16

    @functools.partial(
        pl.kernel,
        mesh=mesh,
        compiler_params=pltpu.CompilerParams(needs_layout_passes=False),
        out_type=(
            jax.ShapeDtypeStruct((B * K,), jnp.float32),
            jax.ShapeDtypeStruct((B * K,), jnp.float32),
        ),
        scratch_types=[
            pltpu.VMEM((b_per_w,), jnp.int32),
            pltpu.VMEM((S1,), jnp.int32),
            pltpu.SMEM((b_per_w,), jnp.int32),
            pltpu.SMEM((S1,), jnp.int32),
            pltpu.VMEM((b_per_w * K,), jnp.float32),
        ]
        + [pltpu.VMEM((K, _W), jnp.float32) for _ in range(_NB)]
        + [pltpu.SemaphoreType.DMA for _ in range(_NB)]
        + [pltpu.SemaphoreType.DMA],
    )
    def gk(vtab_t, vtail, mtab_t, mtail, word, runs,
           ustage, vstage,
           word_v, runs_v, word_s, runs_s, rowbuf,
           *rest):
        bufs = rest[:_NB]
        sems = rest[_NB:2 * _NB]
        sem_o = rest[2 * _NB]
        wid = lax.axis_index("s") * _NC + lax.axis_index("c")
        lanes = lax.iota(jnp.int32, _L)

        def run_table(tab, tail, half, stage):
            pltpu.sync_copy(
                word.at[pl.ds(half * B + wid * b_per_w, b_per_w)], word_v
            )
            pltpu.sync_copy(
                runs.at[pl.ds((half * _NW + wid) * S1, S1)], runs_v
            )

            def fill_w(g, _):
                v = word_v[pl.ds(g * _L, _L)]
                for l in range(_L):
                    word_s[g * _L + l] = v[l]
                return 0

            def fill_r(g, _):
                v = runs_v[pl.ds(g * _L, _L)]
                for l in range(_L):
                    runs_s[g * _L + l] = v[l]
                return 0

            lax.fori_loop(0, b_per_w // _L, fill_w, 0)
            lax.fori_loop(0, S1 // _L, fill_r, 0)
            n = runs_s[S1 - 1]

            def issue(f, s):
                e = runs_s[f]
                blk = e & 0x1FFF

                @pl.when(blk > 0)
                def _():
                    off = pl.multiple_of((blk - 1) * _W, _W)
                    pltpu.async_copy(
                        tab.at[:, pl.ds(off, _W)], bufs[s], sems[s]
                    )

                @pl.when(blk == 0)
                def _():
                    pltpu.async_copy(tail.at[:, pl.ds(0, _W)], bufs[s], sems[s])

            for s in range(_NB):
                @pl.when(s < n)
                def _(s=s):
                    issue(s, s)

            def gbody(g, _):
                for s in range(_NB):
                    f = g * _NB + s

                    @pl.when(f < n)
                    def _(f=f, s=s):
                        pltpu.make_async_copy(
                            tab.at[:, pl.ds(0, _W)], bufs[s], sems[s]
                        ).wait()
                        j0 = runs_s[f] >> 13
                        j1 = runs_s[f + 1] >> 13

                        def ebody(j, _):
                            w = word_s[j]
                            cv = lanes * 0 + (w & 0x7F)
                            g0 = plsc.load_gather(bufs[s], [lanes, cv])
                            g1 = plsc.load_gather(bufs[s], [lanes + _L, cv])
                            rowbuf[pl.ds(j * K, _L)] = g0
                            rowbuf[pl.ds(j * K + _L, _L)] = g1
                            return 0

                        lax.fori_loop(j0, j1, ebody, 0)

                        @pl.when(f + _NB < n)
                        def _():
                            issue(f + _NB, s)
                return 0

            lax.fori_loop(0, (n + _NB - 1) // _NB, gbody, 0)

            def obody(j, _):
                p = word_s[j] >> 16
                pltpu.async_copy(
                    rowbuf.at[pl.ds(j * K, K)],
                    stage.at[pl.ds(p * K, K)],
                    sem_o,
                )
                return 0

            lax.fori_loop(0, b_per_w, obody, 0)
            pltpu.make_async_copy(
                rowbuf, stage.at[pl.ds(0, b_per_w * K)], sem_o
            ).wait()

        run_table(vtab_t, vtail, 0, ustage)
        run_table(mtab_t, mtail, 1, vstage)

    return gk


def _make_dot_kernel(B, K, b_per_w):
    mesh = plsc.VectorSubcoreMesh(core_axis_name="c", subcore_axis_name="s")

    @functools.partial(
        pl.kernel,
        mesh=mesh,
        compiler_params=pltpu.CompilerParams(needs_layout_passes=False),
        out_type=jax.ShapeDtypeStruct((B,), jnp.float32),
        scratch_types=[
            pltpu.VMEM((b_per_w * K,), jnp.float32),
            pltpu.VMEM((b_per_w * K,), jnp.float32),
            pltpu.VMEM((b_per_w * _L,), jnp.float32),
            pltpu.VMEM((b_per_w,), jnp.float32),
        ],
    )
    def dk(ustage, vstage, out_hbm, ubuf, vbuf, partials, outv):
        wid = lax.axis_index("s") * _NC + lax.axis_index("c")
        base = wid * b_per_w
        pltpu.sync_copy(ustage.at[pl.ds(base * K, b_per_w * K)], ubuf)
        pltpu.sync_copy(vstage.at[pl.ds(base * K, b_per_w * K)], vbuf)
        lanes = lax.iota(jnp.int32, _L)

        def row_body(j, _):
            r0 = j * K
            p = ubuf[pl.ds(r0, _L)] * vbuf[pl.ds(r0, _L)]
            p = p + ubuf[pl.ds(r0 + _L, _L)] * vbuf[pl.ds(r0 + _L, _L)]
            partials[pl.ds(j * _L, _L)] = p
            return 0

        lax.fori_loop(0, b_per_w, row_body, 0)

        def red_body(g, _):
            bidx = g * (_L * _L) + lanes * _L
            acc = plsc.load_gather(partials, [bidx])
            for l in range(1, _L):
                acc = acc + plsc.load_gather(partials, [bidx + l])
            outv[pl.ds(g * _L, _L)] = acc
            return 0

        lax.fori_loop(0, b_per_w // _L, red_body, 0)
        pltpu.sync_copy(outv, out_hbm.at[pl.ds(base, b_per_w)])

    return dk


def kernel(viewer_ids, movie_ids, viewer_table, movie_table):
    B = viewer_ids.shape[0]
    K = viewer_table.shape[1]
    NV = viewer_table.shape[0]
    NM = movie_table.shape[0]
    b_per_w = B // _NW

    word, runs = _prep(viewer_ids, movie_ids, NV, NM, b_per_w)

    vt = jnp.swapaxes(viewer_table, 0, 1)
    mt = jnp.swapaxes(movie_table, 0, 1)
    vcut = (NV // _W) * _W
    mcut = (NM // _W) * _W
    vtail = jnp.pad(
        jnp.swapaxes(viewer_table[vcut:], 0, 1), ((0, 0), (0, _W - (NV - vcut)))
    )
    mtail = jnp.pad(
        jnp.swapaxes(movie_table[mcut:], 0, 1), ((0, 0), (0, _W - (NM - mcut)))
    )

    gk = _make_gather_kernel(B, K, b_per_w, NV, NM)
    ustage, vstage = gk(vt, vtail, mt, mtail, word, runs)
    dk = _make_dot_kernel(B, K, b_per_w)
    return dk(ustage, vstage)
